# baseline (speedup 1.0000x reference)
import jax
import jax.numpy as jnp
from jax.experimental import pallas as pl
from jax.experimental.pallas import tpu as pltpu

AUG_T = 1024
B, L, D = 16, 4096, 128
RPB = 4
CH = 512


def _body(lens_ref, x_ref, o_ref, nrows):
    i = pl.program_id(0)

    for r in range(RPB):
        slen = lens_ref[i * RPB + r]
        is_long = slen > AUG_T

        for c in range(L // CH):
            base = c * CH
            masked = is_long & (base < slen)

            @pl.when(masked)
            def _mask(r=r, base=base):
                o_ref[r, pl.ds(base, CH), :] = x_ref[r, pl.ds(base, CH), :]
                first = -(-base // 10) * 10
                for p in range(first, base + CH, 10):
                    o_ref[r, pl.ds(p, 1), :] = jnp.where(
                        p < slen, 0.0, x_ref[r, pl.ds(p, 1), :]
                    )

            @pl.when(jnp.logical_not(masked))
            def _copy(r=r, base=base):
                o_ref[r, pl.ds(base, CH), :] = x_ref[r, pl.ds(base, CH), :]


def _half(seqs, lens, nrows):
    import functools

    return pl.pallas_call(
        functools.partial(_body, nrows=nrows),
        grid=(nrows // RPB,),
        in_specs=[
            pl.BlockSpec(memory_space=pltpu.SMEM),
            pl.BlockSpec((RPB, L, D), lambda i: (i, 0, 0)),
        ],
        out_specs=pl.BlockSpec((RPB, L, D), lambda i: (i, 0, 0)),
        out_shape=jax.ShapeDtypeStruct((nrows, L, D), jnp.float32),
        compiler_params=pltpu.CompilerParams(
            dimension_semantics=("arbitrary",),
        ),
    )(lens, seqs)


def kernel(sequences, seq_lens):
    a = _half(sequences[:8], seq_lens[:8], 8)
    b = _half(sequences[8:], seq_lens[8:], 8)
    return jnp.concatenate([a, b], axis=0), seq_lens


# speedup vs baseline: 1.4536x; 1.4536x over previous
import functools

import jax
import jax.numpy as jnp
from jax import lax
from jax.experimental import pallas as pl
from jax.experimental.pallas import tpu as pltpu
from jax.experimental.pallas import tpu_sc as plsc

AUG_T = 1024
B, L, D = 16, 4096, 128
HALF = L // 2
CPOS = 256
NCH = HALF // CPOS
NBUF = 3

_mesh = plsc.VectorSubcoreMesh(core_axis_name="c", subcore_axis_name="s")


@functools.partial(
    pl.kernel,
    out_type=jax.ShapeDtypeStruct((B, L, D), jnp.float32),
    mesh=_mesh,
    scratch_types=[
        pltpu.VMEM((NBUF, CPOS, D), jnp.float32),
        pltpu.VMEM((32,), jnp.int32),
        pltpu.SemaphoreType.DMA((NBUF,)),
        pltpu.SemaphoreType.DMA((NBUF,)),
    ],
)
def _sc_aug(x_hbm, lens_hbm, o_hbm, buf, lens_v, in_sem, out_sem):
    wid = lax.axis_index("s") * 2 + lax.axis_index("c")
    row = wid // 2
    base0 = (wid % 2) * HALF

    pltpu.sync_copy(lens_hbm, lens_v.at[pl.ds(0, 16)])
    slen = lens_v[pl.ds(row, 16)][0]
    lim_half = jnp.where(slen > AUG_T, jnp.minimum(slen - base0, HALF), 0)

    zeros16 = jnp.zeros((16,), jnp.float32)

    def start_in(k):
        s = k % NBUF
        return pltpu.async_copy(
            x_hbm.at[row, pl.ds(base0 + k * CPOS, CPOS)], buf.at[s], in_sem.at[s]
        )

    def start_out(k):
        s = k % NBUF
        return pltpu.async_copy(
            buf.at[s], o_hbm.at[row, pl.ds(base0 + k * CPOS, CPOS)], out_sem.at[s]
        )

    def zero_chunk(k):
        s = k % NBUF
        cbase = k * CPOS
        first = (10 - (base0 + cbase) % 10) % 10
        limit = jnp.clip(lim_half - cbase, 0, CPOS)
        ntrip = jnp.maximum(0, (limit - first + 9) // 10)

        def zbody(j, _, first=first, s=s):
            p = first + 10 * j
            for i in range(D // 16):
                buf[s, p, pl.ds(16 * i, 16)] = zeros16
            return 0

        lax.fori_loop(0, ntrip, zbody, 0)

    hin = [None] * NCH
    hout = [None] * NCH
    for k in range(NBUF):
        hin[k] = start_in(k)
    for k in range(NCH):
        if k >= 2 and k + 1 < NCH and k + 1 >= NBUF:
            hout[k - 2].wait()
            hin[k + 1] = start_in(k + 1)
        hin[k].wait()
        zero_chunk(k)
        hout[k] = start_out(k)
    hout[NCH - 3].wait()
    hout[NCH - 2].wait()
    hout[NCH - 1].wait()


def kernel(sequences, seq_lens):
    return _sc_aug(sequences, seq_lens), seq_lens


# speedup vs baseline: 2.6363x; 1.8136x over previous
import jax
import jax.numpy as jnp
from jax.experimental import pallas as pl
from jax.experimental.pallas import tpu as pltpu

AUG_T = 1024
B, L, D = 16, 4096, 128
CH = 512
NBUF = 3


def _body(lens_ref, x_ref, o_ref, buf, in_sem, out_sem):
    def start_in(k):
        pltpu.make_async_copy(x_ref.at[k], buf.at[k % NBUF], in_sem.at[k % NBUF]).start()

    def wait_in(k):
        pltpu.make_async_copy(x_ref.at[k], buf.at[k % NBUF], in_sem.at[k % NBUF]).wait()

    def start_out(k):
        pltpu.make_async_copy(buf.at[k % NBUF], o_ref.at[k], out_sem.at[k % NBUF]).start()

    def wait_out(k):
        pltpu.make_async_copy(buf.at[k % NBUF], o_ref.at[k], out_sem.at[k % NBUF]).wait()

    def zero_row(k):
        s = k % NBUF
        slen = lens_ref[k]
        is_long = slen > AUG_T
        for c in range(L // CH):
            base = c * CH

            @pl.when(is_long & (base < slen))
            def _mask(s=s, base=base, slen=slen):
                first = -(-base // 10) * 10
                for p in range(first, base + CH, 10):
                    buf[s, pl.ds(p, 1), :] = jnp.where(
                        p < slen, 0.0, buf[s, pl.ds(p, 1), :]
                    )

    for k in range(NBUF):
        start_in(k)
    for k in range(B):
        if k >= 2 and k + 1 < B and k + 1 >= NBUF:
            wait_out(k - 2)
            start_in(k + 1)
        wait_in(k)
        zero_row(k)
        start_out(k)
    for k in range(B - NBUF, B):
        wait_out(k)


def kernel(sequences, seq_lens):
    out = pl.pallas_call(
        _body,
        in_specs=[
            pl.BlockSpec(memory_space=pltpu.SMEM),
            pl.BlockSpec(memory_space=pl.MemorySpace.ANY),
        ],
        out_specs=pl.BlockSpec(memory_space=pl.MemorySpace.ANY),
        out_shape=jax.ShapeDtypeStruct((B, L, D), jnp.float32),
        scratch_shapes=[
            pltpu.VMEM((NBUF, L, D), jnp.float32),
            pltpu.SemaphoreType.DMA((NBUF,)),
            pltpu.SemaphoreType.DMA((NBUF,)),
        ],
    )(seq_lens, sequences)
    return out, seq_lens


# speedup vs baseline: 2.9469x; 1.1178x over previous
import jax
import jax.numpy as jnp
from jax.experimental import pallas as pl
from jax.experimental.pallas import tpu as pltpu

AUG_T = 1024
B, L, D = 16, 4096, 128
CH = 512
CR = 2
NC = B // CR
NBUF = 4


def _body(lens_ref, x_ref, o_ref, buf, in_sem, out_sem):
    def start_in(k):
        pltpu.make_async_copy(
            x_ref.at[pl.ds(k * CR, CR)], buf.at[k % NBUF], in_sem.at[k % NBUF]
        ).start()

    def wait_in(k):
        pltpu.make_async_copy(
            x_ref.at[pl.ds(k * CR, CR)], buf.at[k % NBUF], in_sem.at[k % NBUF]
        ).wait()

    def start_out(k):
        pltpu.make_async_copy(
            buf.at[k % NBUF], o_ref.at[pl.ds(k * CR, CR)], out_sem.at[k % NBUF]
        ).start()

    def wait_out(k):
        pltpu.make_async_copy(
            buf.at[k % NBUF], o_ref.at[pl.ds(k * CR, CR)], out_sem.at[k % NBUF]
        ).wait()

    def zero_chunk(k):
        s = k % NBUF
        for r in range(CR):
            slen = lens_ref[k * CR + r]
            is_long = slen > AUG_T
            for c in range(L // CH):
                base = c * CH

                @pl.when(is_long & (base < slen))
                def _mask(s=s, r=r, base=base, slen=slen):
                    first = -(-base // 10) * 10
                    for p in range(first, base + CH, 10):
                        buf[s, r, pl.ds(p, 1), :] = jnp.where(
                            p < slen, 0.0, buf[s, r, pl.ds(p, 1), :]
                        )

    for k in range(min(NBUF, NC)):
        start_in(k)
    for k in range(NC):
        if k + 1 < NC and k + 1 >= NBUF:
            wait_out(k + 1 - NBUF)
            start_in(k + 1)
        wait_in(k)
        zero_chunk(k)
        start_out(k)
    for k in range(max(0, NC - NBUF), NC):
        wait_out(k)


def kernel(sequences, seq_lens):
    out = pl.pallas_call(
        _body,
        in_specs=[
            pl.BlockSpec(memory_space=pltpu.SMEM),
            pl.BlockSpec(memory_space=pl.MemorySpace.ANY),
        ],
        out_specs=pl.BlockSpec(memory_space=pl.MemorySpace.ANY),
        out_shape=jax.ShapeDtypeStruct((B, L, D), jnp.float32),
        scratch_shapes=[
            pltpu.VMEM((NBUF, CR, L, D), jnp.float32),
            pltpu.SemaphoreType.DMA((NBUF,)),
            pltpu.SemaphoreType.DMA((NBUF,)),
        ],
    )(seq_lens, sequences)
    return out, seq_lens


# speedup vs baseline: 2.9845x; 1.0128x over previous
import jax
import jax.numpy as jnp
from jax.experimental import pallas as pl
from jax.experimental.pallas import tpu as pltpu

AUG_T = 1024
B, L, D = 16, 4096, 128
CH = 512
CR = 2
NC = B // CR
NBUF = 6


def _body(lens_ref, x_ref, o_ref, buf, in_sem, out_sem):
    def start_in(k):
        pltpu.make_async_copy(
            x_ref.at[pl.ds(k * CR, CR)], buf.at[k % NBUF], in_sem.at[k % NBUF]
        ).start()

    def wait_in(k):
        pltpu.make_async_copy(
            x_ref.at[pl.ds(k * CR, CR)], buf.at[k % NBUF], in_sem.at[k % NBUF]
        ).wait()

    def start_out(k):
        pltpu.make_async_copy(
            buf.at[k % NBUF], o_ref.at[pl.ds(k * CR, CR)], out_sem.at[k % NBUF]
        ).start()

    def wait_out(k):
        pltpu.make_async_copy(
            buf.at[k % NBUF], o_ref.at[pl.ds(k * CR, CR)], out_sem.at[k % NBUF]
        ).wait()

    def zero_chunk(k):
        s = k % NBUF
        for r in range(CR):
            slen = lens_ref[k * CR + r]
            is_long = slen > AUG_T
            for c in range(L // CH):
                base = c * CH

                @pl.when(is_long & (base < slen))
                def _mask(s=s, r=r, base=base, slen=slen):
                    first = -(-base // 10) * 10
                    for p in range(first, base + CH, 10):
                        buf[s, r, pl.ds(p, 1), :] = jnp.where(
                            p < slen, 0.0, buf[s, r, pl.ds(p, 1), :]
                        )

    for k in range(min(NBUF, NC)):
        start_in(k)
    for k in range(NC):
        if k + 1 < NC and k + 1 >= NBUF:
            wait_out(k + 1 - NBUF)
            start_in(k + 1)
        wait_in(k)
        zero_chunk(k)
        start_out(k)
    for k in range(max(0, NC - NBUF), NC):
        wait_out(k)


def kernel(sequences, seq_lens):
    out = pl.pallas_call(
        _body,
        in_specs=[
            pl.BlockSpec(memory_space=pltpu.SMEM),
            pl.BlockSpec(memory_space=pl.MemorySpace.ANY),
        ],
        out_specs=pl.BlockSpec(memory_space=pl.MemorySpace.ANY),
        out_shape=jax.ShapeDtypeStruct((B, L, D), jnp.float32),
        scratch_shapes=[
            pltpu.VMEM((NBUF, CR, L, D), jnp.float32),
            pltpu.SemaphoreType.DMA((NBUF,)),
            pltpu.SemaphoreType.DMA((NBUF,)),
        ],
    )(seq_lens, sequences)
    return out, seq_lens
